# Initial kernel scaffold; baseline (speedup 1.0000x reference)
import jax
import jax.numpy as jnp
from jax.experimental import pallas as pl


def kernel(x, edge_index, W_proj, b_proj, att_src, att_dst, W_k, b_k, q, bn1_g, bn1_b, bn2_g, bn2_b, fc1_W, fc1_b, bn3_g, bn3_b, fc2_W, fc2_b, fc3_W, fc3_b, bn4_g, bn4_b):
    raise NotImplementedError("write your pallas kernel here")


# speedup vs baseline: 69.8637x; 69.8637x over previous
import functools

import jax
import jax.numpy as jnp
from jax import lax
from jax.experimental import pallas as pl
from jax.experimental.pallas import tpu as pltpu
from jax.experimental.pallas import tpu_sc as plsc

N = 10000
E = 320000
D_IN = 128
HID = 128
H = 8
C = 16

CHUNK = 128
NCHUNK = E // CHUNK
NW = 32
NPAD = 10240
ROWS_PER_TILE = NPAD // 16


def _pre_body(x_ref, w_ref, b_ref, asrc_w_ref, adst_w_ref,
              h_ref, asrc_ref, adst_ref, m_ref):
    h = jnp.dot(x_ref[...], w_ref[...], preferred_element_type=jnp.float32)
    h = h + b_ref[...]
    h_ref[...] = h
    asrc = jnp.dot(h, asrc_w_ref[...], preferred_element_type=jnp.float32)
    adst = jnp.dot(h, adst_w_ref[...], preferred_element_type=jnp.float32)
    asrc_ref[...] = asrc
    adst_ref[...] = adst
    msum = (jnp.max(asrc, axis=0, keepdims=True)
            + jnp.max(adst, axis=0, keepdims=True))
    m_ref[...] = jnp.maximum(msum, 0.0)


def _pre_pass(x, w_proj, b_proj, asrc_w, adst_w):
    return pl.pallas_call(
        _pre_body,
        out_shape=(
            jax.ShapeDtypeStruct((N, HID), jnp.float32),
            jax.ShapeDtypeStruct((N, 16), jnp.float32),
            jax.ShapeDtypeStruct((N, 16), jnp.float32),
            jax.ShapeDtypeStruct((1, 16), jnp.float32),
        ),
    )(x, w_proj, b_proj, asrc_w, adst_w)


def _sc_edge_body(row_hbm, col_hbm, h_hbm, asrc_hbm, adst_hbm, m_hbm,
                  zmsg_hbm, zex_hbm,
                  outm_hbm, outex_hbm,
                  idxr_v, idxc_v, hrow_v, asr_v, adc_v, m_v,
                  sem0, sem1, sem2,
                  accm_sh, accex_sh):
    cid = lax.axis_index("c")
    sid = lax.axis_index("s")
    wid = sid * 2 + cid

    zbase = sid * ROWS_PER_TILE
    pltpu.sync_copy(zmsg_hbm, accm_sh.at[pl.ds(zbase, ROWS_PER_TILE)])
    pltpu.sync_copy(zex_hbm, accex_sh.at[pl.ds(zbase, ROWS_PER_TILE)])
    pltpu.sync_copy(m_hbm.at[0], m_v)
    plsc.subcore_barrier()

    mvec = m_v[...]

    def chunk_body(g, carry):
        chunk = wid + g * NW

        @pl.when(chunk < NCHUNK)
        def _():
            base = pl.multiple_of(chunk * CHUNK, 8)
            pltpu.sync_copy(row_hbm.at[pl.ds(base, CHUNK)], idxr_v)
            pltpu.sync_copy(col_hbm.at[pl.ds(base, CHUNK)], idxc_v)
            cp0 = pltpu.async_copy(h_hbm.at[idxr_v], hrow_v, sem0)
            cp1 = pltpu.async_copy(asrc_hbm.at[idxr_v], asr_v, sem1)
            cp2 = pltpu.async_copy(adst_hbm.at[idxc_v], adc_v, sem2)
            cp1.wait()
            cp2.wait()

            def edge_w(j, carry2):
                av = asr_v[j] + adc_v[j]
                lv = jnp.where(av >= 0.0, av, 0.2 * av)
                asr_v[j] = jnp.exp(lv - mvec)
                return carry2

            lax.fori_loop(0, CHUNK, edge_w, 0, unroll=4)
            cp0.wait()

            def edge_m(j, carry2):
                ev = asr_v[j]
                for hh in range(H):
                    e = lax.broadcast(ev[hh], (16,))
                    hrow_v[j, pl.ds(hh * 16, 16)] = (
                        hrow_v[j, pl.ds(hh * 16, 16)] * e)
                return carry2

            lax.fori_loop(0, CHUNK, edge_m, 0, unroll=2)

            pltpu.sync_copy(hrow_v, accm_sh.at[idxc_v], add=True)
            pltpu.sync_copy(asr_v, accex_sh.at[idxc_v], add=True)

        return carry

    nloc = (NCHUNK + NW - 1) // NW
    lax.fori_loop(0, nloc, chunk_body, 0)

    plsc.subcore_barrier()
    pltpu.sync_copy(accm_sh.at[pl.ds(zbase, ROWS_PER_TILE)],
                    outm_hbm.at[cid, pl.ds(zbase, ROWS_PER_TILE)])
    pltpu.sync_copy(accex_sh.at[pl.ds(zbase, ROWS_PER_TILE)],
                    outex_hbm.at[cid, pl.ds(zbase, ROWS_PER_TILE)])


def _sc_edge_pass(row, col, h, asrc, adst, m, zmsg, zex):
    mesh = plsc.VectorSubcoreMesh(core_axis_name="c", subcore_axis_name="s")
    f = functools.partial(
        pl.kernel,
        mesh=mesh,
        compiler_params=pltpu.CompilerParams(use_tc_tiling_on_sc=False),
        out_type=(
            jax.ShapeDtypeStruct((2, NPAD, HID), jnp.float32),
            jax.ShapeDtypeStruct((2, NPAD, 16), jnp.float32),
        ),
        scratch_types=[
            pltpu.VMEM((CHUNK,), jnp.int32),
            pltpu.VMEM((CHUNK,), jnp.int32),
            pltpu.VMEM((CHUNK, HID), jnp.float32),
            pltpu.VMEM((CHUNK, 16), jnp.float32),
            pltpu.VMEM((CHUNK, 16), jnp.float32),
            pltpu.VMEM((16,), jnp.float32),
            pltpu.SemaphoreType.DMA,
            pltpu.SemaphoreType.DMA,
            pltpu.SemaphoreType.DMA,
            pltpu.VMEM_SHARED((NPAD, HID), jnp.float32),
            pltpu.VMEM_SHARED((NPAD, 16), jnp.float32),
        ],
    )(_sc_edge_body)
    return f(row, col, h, asrc, adst, m, zmsg, zex)


def _bn_relu(x, g, b):
    m = jnp.mean(x, axis=0, keepdims=True)
    d = x - m
    v = jnp.mean(d * d, axis=0, keepdims=True)
    return jnp.maximum(g * d * lax.rsqrt(v + 1e-5) + b, 0.0)


def _post_body(outm_ref, outex_ref, expand_ref, pool_ref,
               bn1_g_ref, bn1_b_ref, bn2_g_ref, bn2_b_ref,
               fc1_W_ref, fc1_b_ref, bn3_g_ref, bn3_b_ref,
               fc2_W_ref, fc2_b_ref, fc3_W_ref, fc3_b_ref,
               bn4_g_ref, bn4_b_ref, out_ref):
    num = outm_ref[0, :N] + outm_ref[1, :N]
    den = outex_ref[0, :N] + outex_ref[1, :N]
    recip = 1.0 / (den + 1e-16)
    recip_e = jnp.dot(recip, expand_ref[...],
                      preferred_element_type=jnp.float32)
    out = jnp.maximum(num * recip_e, 0.0)

    xh = _bn_relu(out, bn1_g_ref[...], bn1_b_ref[...])
    xh = jnp.dot(xh, pool_ref[...], preferred_element_type=jnp.float32)
    xh = _bn_relu(xh, bn2_g_ref[...], bn2_b_ref[...])
    xh = jnp.dot(xh, fc1_W_ref[...],
                 preferred_element_type=jnp.float32) + fc1_b_ref[...]
    xh = _bn_relu(xh, bn3_g_ref[...], bn3_b_ref[...])
    xh = jnp.dot(xh, fc2_W_ref[...],
                 preferred_element_type=jnp.float32) + fc2_b_ref[...]
    xh = _bn_relu(xh, bn3_g_ref[...], bn3_b_ref[...])
    xh = jnp.dot(xh, fc3_W_ref[...],
                 preferred_element_type=jnp.float32) + fc3_b_ref[...]
    out_ref[...] = _bn_relu(xh, bn4_g_ref[...], bn4_b_ref[...])


def _post_pass(outm, outex, expand, pool, bn1_g, bn1_b, bn2_g, bn2_b,
               fc1_W, fc1_b, bn3_g, bn3_b, fc2_W, fc2_b, fc3_W, fc3_b,
               bn4_g, bn4_b):
    return pl.pallas_call(
        _post_body,
        out_shape=jax.ShapeDtypeStruct((N, 64), jnp.float32),
    )(outm, outex, expand, pool, bn1_g, bn1_b, bn2_g, bn2_b,
      fc1_W, fc1_b, bn3_g, bn3_b, fc2_W, fc2_b, fc3_W, fc3_b,
      bn4_g, bn4_b)


def kernel(x, edge_index, W_proj, b_proj, att_src, att_dst, W_k, b_k, q,
           bn1_g, bn1_b, bn2_g, bn2_b, fc1_W, fc1_b, bn3_g, bn3_b,
           fc2_W, fc2_b, fc3_W, fc3_b, bn4_g, bn4_b):
    del W_k, b_k, q

    rows = jnp.arange(HID, dtype=jnp.int32)
    heads = rows // C
    asrc_w = jnp.zeros((HID, 16), jnp.float32).at[rows, heads].set(
        att_src.reshape(-1))
    adst_w = jnp.zeros((HID, 16), jnp.float32).at[rows, heads].set(
        att_dst.reshape(-1))
    expand = jnp.zeros((16, HID), jnp.float32).at[heads, rows].set(1.0)
    pool = jnp.zeros((HID, HID // 2), jnp.float32).at[
        jnp.arange(HID), jnp.arange(HID) // 2].set(0.5)

    row = edge_index[0].astype(jnp.int32)
    col = edge_index[1].astype(jnp.int32)

    h, asrc, adst, m = _pre_pass(x, W_proj, b_proj.reshape(1, HID),
                                 asrc_w, adst_w)

    zmsg = jnp.zeros((ROWS_PER_TILE, HID), jnp.float32)
    zex = jnp.zeros((ROWS_PER_TILE, 16), jnp.float32)
    outm, outex = _sc_edge_pass(row, col, h, asrc, adst, m, zmsg, zex)

    return _post_pass(
        outm, outex, expand, pool,
        bn1_g.reshape(1, -1), bn1_b.reshape(1, -1),
        bn2_g.reshape(1, -1), bn2_b.reshape(1, -1),
        fc1_W, fc1_b.reshape(1, -1),
        bn3_g.reshape(1, -1), bn3_b.reshape(1, -1),
        fc2_W, fc2_b.reshape(1, -1),
        fc3_W, fc3_b.reshape(1, -1),
        bn4_g.reshape(1, -1), bn4_b.reshape(1, -1))
